# Initial kernel scaffold; baseline (speedup 1.0000x reference)
import jax
import jax.numpy as jnp
from jax.experimental import pallas as pl


def kernel(x, edge_index, W1, b1, W2, b2, Wg, bg, Wa, ba):
    raise NotImplementedError("write your pallas kernel here")


# speedup vs baseline: 6.1830x; 6.1830x over previous
import functools

import jax
import jax.numpy as jnp
from jax import lax
from jax.experimental import pallas as pl
from jax.experimental.pallas import tpu as pltpu
from jax.experimental.pallas import tpu_sc as plsc

_N, _D, _E = 10000, 128, 320000
_CHUNK = 128


def _sc_scatter_add(x, src, dst):
    info = plsc.get_sparse_core_info()
    nc, ns = info.num_cores, info.num_subcores
    nw = nc * ns
    n_chunks = _E // _CHUNK
    assert _E % _CHUNK == 0
    rows_per_tile = 640
    n_pad = rows_per_tile * ns
    zp = rows_per_tile // _CHUNK
    zrows = _CHUNK
    assert rows_per_tile % _CHUNK == 0

    mesh = plsc.VectorSubcoreMesh(core_axis_name="c", subcore_axis_name="s")

    @functools.partial(
        pl.kernel,
        out_type=jax.ShapeDtypeStruct((nc, n_pad, _D), jnp.float32),
        mesh=mesh,
        scratch_types=[
            pltpu.MemorySpace.VMEM_SHARED((n_pad, _D), jnp.float32),
            pltpu.MemorySpace.VMEM((_CHUNK,), jnp.int32),
            pltpu.MemorySpace.VMEM((_CHUNK,), jnp.int32),
            pltpu.MemorySpace.VMEM((_CHUNK, _D), jnp.float32),
            pltpu.SemaphoreType.DMA,
        ],
    )
    def k(x_hbm, src_hbm, dst_hbm, out_hbm, agg_sh, src_v, dst_v, rows_v, sem):
        c = lax.axis_index("c")
        s = lax.axis_index("s")
        wid = s * nc + c

        def zrow(i, _):
            def zlane(j, _):
                rows_v[i, pl.ds(j * 16, 16)] = jnp.zeros((16,), jnp.float32)
                return 0

            lax.fori_loop(0, _D // 16, zlane, 0)
            return 0

        lax.fori_loop(0, zrows, zrow, 0)

        for p in range(zp):
            pltpu.sync_copy(
                rows_v.at[pl.ds(0, zrows)],
                agg_sh.at[pl.ds(s * rows_per_tile + p * zrows, zrows)],
            )
        plsc.subcore_barrier()

        base_count = n_chunks // nw
        extra = n_chunks % nw
        my_count = base_count + jnp.where(wid < extra, 1, 0)

        def body(i, _):
            chunk = i * nw + wid
            base = chunk * _CHUNK
            pltpu.sync_copy(src_hbm.at[pl.ds(base, _CHUNK)], src_v)
            pltpu.sync_copy(dst_hbm.at[pl.ds(base, _CHUNK)], dst_v)
            pltpu.async_copy(x_hbm.at[src_v], rows_v, sem).wait()
            pltpu.sync_copy(rows_v, agg_sh.at[dst_v], add=True)
            return 0

        lax.fori_loop(0, my_count, body, 0)
        plsc.subcore_barrier()

        pltpu.sync_copy(
            agg_sh.at[pl.ds(s * rows_per_tile, rows_per_tile)],
            out_hbm.at[c, pl.ds(s * rows_per_tile, rows_per_tile)],
        )

    return k(x, src, dst)


def _mlp_body(x_ref, p_ref, w1_ref, b1_ref, w2_ref, b2_ref, wg_ref, bg_ref,
              wa_ref, ba_ref, h_ref, a_ref):
    dn = (((1,), (1,)), ((), ()))
    xa = x_ref[...] + p_ref[0, :_N] + p_ref[1, :_N]
    h1 = lax.dot_general(xa, w1_ref[...], dn, preferred_element_type=jnp.float32)
    h1 = jnp.maximum(h1 + b1_ref[...], 0.0)
    h = lax.dot_general(h1, w2_ref[...], dn, preferred_element_type=jnp.float32)
    h = h + b2_ref[...]
    h_ref[...] = h
    ga = lax.dot_general(h, wg_ref[...], dn, preferred_element_type=jnp.float32)
    ga = jnp.tanh(ga + bg_ref[...])
    alpha = lax.dot_general(wa_ref[...], ga, dn, preferred_element_type=jnp.float32)
    e = jnp.exp(alpha - jnp.max(alpha))
    a_ref[...] = e / jnp.sum(e)


def kernel(x, edge_index, W1, b1, W2, b2, Wg, bg, Wa, ba):
    src = edge_index[0]
    dst = edge_index[1]
    parts = _sc_scatter_add(x, src, dst)
    h, a = pl.pallas_call(
        _mlp_body,
        out_shape=[
            jax.ShapeDtypeStruct((_N, _D), jnp.float32),
            jax.ShapeDtypeStruct((1, _N), jnp.float32),
        ],
    )(x, parts, W1, b1.reshape(1, -1), W2, b2.reshape(1, -1),
      Wg, bg.reshape(1, -1), Wa, ba.reshape(1, 1))
    return h, a[0]
